# baseline (speedup 1.0000x reference)
import functools

import jax
import jax.numpy as jnp
from jax import lax
from jax.experimental import pallas as pl
from jax.experimental.pallas import tpu as pltpu
from jax.experimental.pallas import tpu_sc as plsc

NUM_TYPES = 100
NUM_STATES = 100
EMBED_DIM = 32
H1 = 64
H2 = 32
BATCH = 16384

_NC, _NS, _L = 2, 16, 16
_NW = _NC * _NS
_BPW = BATCH // _NW
_GCHUNK = 128
_NGATHER = _BPW // _GCHUNK


def _sc_gather(x0, x1, x2, table):
    mesh = plsc.VectorSubcoreMesh(core_axis_name="c", subcore_axis_name="s")

    @functools.partial(
        pl.kernel,
        mesh=mesh,
        compiler_params=pltpu.CompilerParams(use_tc_tiling_on_sc=False),
        out_type=jax.ShapeDtypeStruct((BATCH, EMBED_DIM), jnp.float32),
        scratch_types=[
            pltpu.VMEM((_BPW,), jnp.int32),
            pltpu.VMEM((_BPW,), jnp.int32),
            pltpu.VMEM((_BPW,), jnp.int32),
            pltpu.VMEM((_BPW,), jnp.int32),
            pltpu.VMEM((_BPW, EMBED_DIM), jnp.float32),
            pltpu.SemaphoreType.DMA,
        ],
    )
    def k(x0_hbm, x1_hbm, x2_hbm, table_hbm, emb_hbm, x0_v, x1_v, x2_v,
          idx_v, rows_v, sem):
        wid = lax.axis_index("s") * _NC + lax.axis_index("c")
        base = wid * _BPW
        pltpu.sync_copy(x0_hbm.at[pl.ds(base, _BPW)], x0_v)
        pltpu.sync_copy(x1_hbm.at[pl.ds(base, _BPW)], x1_v)
        pltpu.sync_copy(x2_hbm.at[pl.ds(base, _BPW)], x2_v)

        for j in range(_BPW // _L):
            sl = pl.ds(j * _L, _L)
            idx_v[sl] = (
                x0_v[sl]
                + x1_v[sl] * NUM_TYPES
                + x2_v[sl] * (NUM_TYPES * NUM_STATES)
            )

        copies = []
        for g in range(_NGATHER):
            copies.append(
                pltpu.async_copy(
                    table_hbm.at[idx_v.at[pl.ds(g * _GCHUNK, _GCHUNK)]],
                    rows_v.at[pl.ds(g * _GCHUNK, _GCHUNK), :],
                    sem,
                )
            )
        for c in copies:
            c.wait()

        pltpu.sync_copy(rows_v, emb_hbm.at[pl.ds(base, _BPW), :])

    return k(x0, x1, x2, table)


def _mlp_body(emb_ref, w1_ref, b1_ref, w2_ref, b2_ref, out_ref):
    h = jnp.dot(emb_ref[...], w1_ref[...], preferred_element_type=jnp.float32)
    h = jnp.maximum(h + b1_ref[...], 0.0)
    o = jnp.dot(h, w2_ref[...], preferred_element_type=jnp.float32)
    out_ref[...] = jnp.maximum(o + b2_ref[...], 0.0)


def _tc_mlp(emb, w1, b1, w2, b2):
    blk = 2048
    grid = (BATCH // blk,)
    return pl.pallas_call(
        _mlp_body,
        grid=grid,
        in_specs=[
            pl.BlockSpec((blk, EMBED_DIM), lambda i: (i, 0)),
            pl.BlockSpec((EMBED_DIM, H1), lambda i: (0, 0)),
            pl.BlockSpec((1, H1), lambda i: (0, 0)),
            pl.BlockSpec((H1, H2), lambda i: (0, 0)),
            pl.BlockSpec((1, H2), lambda i: (0, 0)),
        ],
        out_specs=pl.BlockSpec((blk, H2), lambda i: (i, 0)),
        out_shape=jax.ShapeDtypeStruct((BATCH, H2), jnp.float32),
    )(emb, w1, b1, w2, b2)


def kernel(x, table, W1, b1, W2, b2):
    xi = x.astype(jnp.int32)
    emb = _sc_gather(xi[:, 0], xi[:, 1], xi[:, 2], table)
    return _tc_mlp(emb, W1, b1.reshape(1, H1), W2, b2.reshape(1, H2))
